# baseline (speedup 1.0000x reference)
import jax
import jax.numpy as jnp
from jax import lax
from jax.experimental import pallas as pl
from jax.experimental.pallas import tpu as pltpu
from jax.experimental.pallas import tpu_sc as plsc

D_MODEL = 128
SEQ_LEN = 200
BATCH = 1024

HALF = SEQ_LEN // 2
LANES = 16
VPR = D_MODEL // LANES
SCALE = float(D_MODEL) ** 0.5
NBUF = 3


NSTREAM = 4
SUB = SEQ_LEN // NSTREAM


def _gather(table_hbm, idx, rows, sem):
  for t in range(NSTREAM):
    pltpu.async_copy(
        table_hbm.at[idx.at[t]], rows.at[pl.ds(t * SUB, SUB)], sem)


def _wait_gather(table_hbm, idx, rows, sem):
  for t in range(NSTREAM):
    pltpu.make_async_copy(
        table_hbm.at[idx.at[t]], rows.at[pl.ds(t * SUB, SUB)], sem).wait()


def _body(ids_hbm, table_hbm, pos_hbm, out_hbm,
          idx0, idx1, idx2, rows0, rows1, rows2, pos_v,
          si0, si1, si2, sg0, sg1, sg2, so0, so1, so2):
  info = plsc.get_sparse_core_info()
  nc = info.num_cores
  wid = lax.axis_index("s") * nc + lax.axis_index("c")
  per_w = BATCH // (nc * info.num_subcores)
  base = wid * per_w

  pltpu.sync_copy(pos_hbm, pos_v)

  idxs = (idx0, idx1, idx2)
  rows = (rows0, rows1, rows2)
  sidx = (si0, si1, si2)
  sgat = (sg0, sg1, sg2)
  sout = (so0, so1, so2)

  def wait_idx(b):
    pltpu.make_async_copy(ids_hbm.at[base], idxs[b], sidx[b]).wait()

  pltpu.async_copy(ids_hbm.at[base], idx0, si0)
  pltpu.async_copy(ids_hbm.at[base + 1], idx1, si1)
  wait_idx(0)
  _gather(table_hbm, idx0, rows0, sg0)

  def compute(buf):
    @plsc.parallel_loop(0, SEQ_LEN, unroll=4)
    def row_body(r):
      for k in range(VPR):
        sl = pl.ds(k * LANES, LANES)
        buf[r, sl] = buf[r, sl] * SCALE + pos_v[r, sl]

  def outer(i, carry):
    for b in range(NBUF):
      j = NBUF * i + b

      @pl.when(j < per_w)
      def _slot():
        bn = (b + 1) % NBUF
        bn2 = (b + 2) % NBUF

        @pl.when(j + 1 < per_w)
        def _prefetch():
          wait_idx(bn)

          @pl.when(j >= 2)
          def _drain():
            pltpu.make_async_copy(
                rows[bn],
                out_hbm.at[pl.ds((base + j - 2) * SEQ_LEN, SEQ_LEN)],
                sout[bn]).wait()

          _gather(table_hbm, idxs[bn], rows[bn], sgat[bn])

          @pl.when(j + 2 < per_w)
          def _idx_prefetch():
            pltpu.async_copy(ids_hbm.at[base + j + 2], idxs[bn2], sidx[bn2])

        _wait_gather(table_hbm, idxs[b], rows[b], sgat[b])
        compute(rows[b])
        pltpu.async_copy(
            rows[b], out_hbm.at[pl.ds((base + j) * SEQ_LEN, SEQ_LEN)],
            sout[b])
    return carry

  lax.fori_loop(0, (per_w + NBUF - 1) // NBUF, outer, None)

  for t in range(NBUF):
    j = per_w - NBUF + t
    pltpu.make_async_copy(
        rows[j % NBUF],
        out_hbm.at[pl.ds((base + j) * SEQ_LEN, SEQ_LEN)],
        sout[j % NBUF]).wait()


@jax.jit
def kernel(input_ids, token_table, pos_table):
  batch, seq_len = input_ids.shape
  ids3d = input_ids.reshape(batch, NSTREAM, SUB).astype(jnp.int32)
  pos = pos_table[:seq_len]

  mesh = plsc.VectorSubcoreMesh(core_axis_name="c", subcore_axis_name="s")
  out = pl.kernel(
      _body,
      out_type=jax.ShapeDtypeStruct((batch * seq_len, D_MODEL), jnp.float32),
      mesh=mesh,
      scratch_types=[
          pltpu.VMEM((NSTREAM, SUB), jnp.int32),
          pltpu.VMEM((NSTREAM, SUB), jnp.int32),
          pltpu.VMEM((NSTREAM, SUB), jnp.int32),
          pltpu.VMEM((SEQ_LEN, D_MODEL), jnp.float32),
          pltpu.VMEM((SEQ_LEN, D_MODEL), jnp.float32),
          pltpu.VMEM((SEQ_LEN, D_MODEL), jnp.float32),
          pltpu.VMEM((SEQ_LEN, D_MODEL), jnp.float32),
          pltpu.SemaphoreType.DMA,
          pltpu.SemaphoreType.DMA,
          pltpu.SemaphoreType.DMA,
          pltpu.SemaphoreType.DMA,
          pltpu.SemaphoreType.DMA,
          pltpu.SemaphoreType.DMA,
          pltpu.SemaphoreType.DMA,
          pltpu.SemaphoreType.DMA,
          pltpu.SemaphoreType.DMA,
      ],
  )(ids3d, token_table, pos)
  return out.reshape(batch, seq_len, D_MODEL)
